# Initial kernel scaffold; baseline (speedup 1.0000x reference)
import jax
import jax.numpy as jnp
from jax.experimental import pallas as pl


def kernel(x_enc, rev_w, rev_b, W_ep, b_ep, Wt, bt, W_sp, b_sp, W_ve, b_ve, Wv, bv, W_fc, b_fc, W_fc2, b_fc2, W_fc3, b_fc3):
    raise NotImplementedError("write your pallas kernel here")


# speedup vs baseline: 5.9633x; 5.9633x over previous
import functools
import math

import jax
import jax.numpy as jnp
import numpy as np
from jax import lax
from jax.experimental import pallas as pl

B = 16
L = 720
N = 862
NP = 896
NT = 128
NTILES = 7
D2 = 64
DM = 128
PN = 45
PLEN = 16
PRED = 96
EPS = 1e-5


def _dot(a, b, ca, cb):
    return lax.dot_general(a, b, (((ca,), (cb,)), ((), ())),
                           preferred_element_type=jnp.float32)


def _top3_softmax(s):
    m1 = jnp.max(s, axis=1, keepdims=True)
    s1 = jnp.where(s >= m1, jnp.float32(-1e30), s)
    m2 = jnp.max(s1, axis=1, keepdims=True)
    s2 = jnp.where(s1 >= m2, jnp.float32(-1e30), s1)
    m3 = jnp.max(s2, axis=1, keepdims=True)
    sm = jnp.where(s >= m3, s, jnp.float32(-1e9))
    e = jnp.exp(sm - m1)
    return e / jnp.sum(e, axis=1, keepdims=True)


def _ka_body(x_ref, rw_ref, rb_ref, wep_ref, pe2_ref, psel_ref,
             mean_ref, std_ref, at_ref):
    xb = x_ref[0]
    col = lax.broadcasted_iota(jnp.int32, (L, NP), 1)
    xb = jnp.where(col < N, xb, 0.0)
    mean = jnp.sum(xb, axis=0, keepdims=True) * (1.0 / L)
    sq = jnp.sum(xb * xb, axis=0, keepdims=True) * (1.0 / L)
    var = sq - mean * mean
    std = jnp.sqrt(var + EPS)
    mean_ref[...] = mean[None]
    std_ref[...] = std[None]
    rw = rw_ref[0]
    rb = rb_ref[0]
    c = rw / (std * N)
    colv = lax.broadcasted_iota(jnp.int32, (1, NP), 1)
    c = jnp.where(colv < N, c, 0.0)
    mcol = jnp.sum(xb * c, axis=1, keepdims=True)
    off = jnp.sum(rb * jnp.where(colv < N, 1.0, 0.0)) * (1.0 / N) \
        - jnp.sum(mean * c)
    z = mcol + off
    zg = z * wep_ref[...]
    q = _dot(psel_ref[...], zg, 1, 0) + pe2_ref[...]
    s = _dot(q, q, 1, 1) * (1.0 / math.sqrt(D2))
    at_ref[...] = _top3_softmax(s)[None]


def _kb_body(x_ref, mean_ref, std_ref, rw_ref, rb_ref, at_ref,
             wep_ref, pe2_ref, wt0_ref, bt0_ref, wt1_ref, bt1_ref,
             wsp_ref, bsp_ref, wfc_ref, bfc_ref, wve_ref, bve_ref,
             dt_ref, ev_ref):
    j = pl.program_id(1)
    xb = x_ref[0]
    col = lax.broadcasted_iota(jnp.int32, (L, NT), 1) + j * NT
    mean = mean_ref[0]
    std = std_ref[0]
    xn = (xb - mean) / std * rw_ref[0] + rb_ref[0]
    xn = jnp.where(col < N, xn, 0.0)
    ev = _dot(xn, wve_ref[...], 0, 0) + bve_ref[...]
    ev_ref[...] = ev[None]
    x3 = xn.reshape(PN, PLEN, NT)
    pe2 = pe2_ref[...]
    vs = []
    for p in range(PN):
        vs.append(_dot(wep_ref[...], x3[p], 0, 0) + pe2[p][:, None])
    v = jnp.stack(vs)
    a = at_ref[0]
    for wt, bt in ((wt0_ref, bt0_ref), (wt1_ref, bt1_ref)):
        av = _dot(a, v.reshape(PN, D2 * NT), 1, 0).reshape(PN, D2, NT)
        wtm = wt[...]
        btm = bt[...]
        us = []
        for p in range(PN):
            us.append(jax.nn.gelu(_dot(wtm, av[p], 0, 0) + btm[0][:, None]))
        v = jnp.stack(us)
    acc = jnp.zeros((NT, DM), jnp.float32)
    wsp = wsp_ref[...]
    for p in range(PN):
        acc = acc + _dot(v[p], wsp[p], 0, 0)
    dec_time = acc + bsp_ref[...]
    dt = _dot(dec_time, wfc_ref[...], 1, 0) + bfc_ref[...]
    dt_ref[...] = dt[None]


def _kc_body(ev_ref, dt_ref, wv0_ref, bv0_ref, wv1_ref, bv1_ref,
             wfc2_ref, bfc2_ref, w3a_ref, w3b_ref, bfc3_ref,
             mean_ref, std_ref, rw_ref, rb_ref, out_ref):
    ev = ev_ref[0]
    rows = []
    for i in range(NTILES):
        evc = ev[i * NT:(i + 1) * NT]
        sc = _dot(evc, ev, 1, 1) * (1.0 / math.sqrt(DM))
        colm = lax.broadcasted_iota(jnp.int32, (NT, NP), 1)
        sc = jnp.where(colm < N, sc, jnp.float32(-1e9))
        rows.append(_top3_softmax(sc))
    av = jnp.concatenate(rows, axis=0)
    h = ev
    for wv, bv in ((wv0_ref, bv0_ref), (wv1_ref, bv1_ref)):
        h = jax.nn.gelu(_dot(_dot(av, h, 1, 0), wv[...], 1, 0) + bv[...])
    dv = _dot(h, wfc2_ref[...], 1, 0) + bfc2_ref[...]
    dt = dt_ref[0]
    dec = _dot(dt, w3a_ref[...], 1, 0) + _dot(dv, w3b_ref[...], 1, 0) \
        + bfc3_ref[...]
    mean = mean_ref[0, 0][:, None]
    std = std_ref[0, 0][:, None]
    rw = rw_ref[0, 0][:, None]
    rb = rb_ref[0, 0][:, None]
    dec = (dec - rb) / (rw + EPS * EPS) * std + mean
    out_ref[...] = jnp.transpose(dec)[None]


def _pe2_const():
    pos = np.arange(PN)[:, None].astype(np.float32)
    div = np.exp(np.arange(0, D2, 2).astype(np.float32) * -(np.log(10000.0) / D2))
    pe = np.zeros((PN, D2), dtype=np.float32)
    pe[:, 0::2] = np.sin(pos * div)
    pe[:, 1::2] = np.cos(pos * div)
    return pe


@jax.jit
def kernel(x_enc, rev_w, rev_b, W_ep, b_ep, Wt, bt, W_sp, b_sp, W_ve, b_ve,
           Wv, bv, W_fc, b_fc, W_fc2, b_fc2, W_fc3, b_fc3):
    f32 = jnp.float32
    rw = jnp.pad(rev_w, (0, NP - N)).reshape(1, 1, NP)
    rb = jnp.pad(rev_b, (0, NP - N)).reshape(1, 1, NP)
    pe2 = jnp.asarray(_pe2_const()) + b_ep[None, :]
    wep_tiled = jnp.tile(W_ep, (PN, 1))
    psel = jnp.asarray(np.kron(np.eye(PN, dtype=np.float32),
                               np.ones((1, PLEN), np.float32)))
    wsp3 = W_sp.reshape(PN, D2, DM)
    w3a, w3b = W_fc3[:PRED], W_fc3[PRED:]

    mean, std, a_t = pl.pallas_call(
        _ka_body,
        grid=(B,),
        in_specs=[
            pl.BlockSpec((1, L, NP), lambda b: (b, 0, 0)),
            pl.BlockSpec((1, 1, NP), lambda b: (0, 0, 0)),
            pl.BlockSpec((1, 1, NP), lambda b: (0, 0, 0)),
            pl.BlockSpec((L, D2), lambda b: (0, 0)),
            pl.BlockSpec((PN, D2), lambda b: (0, 0)),
            pl.BlockSpec((PN, L), lambda b: (0, 0)),
        ],
        out_specs=[
            pl.BlockSpec((1, 1, NP), lambda b: (b, 0, 0)),
            pl.BlockSpec((1, 1, NP), lambda b: (b, 0, 0)),
            pl.BlockSpec((1, PN, PN), lambda b: (b, 0, 0)),
        ],
        out_shape=[
            jax.ShapeDtypeStruct((B, 1, NP), f32),
            jax.ShapeDtypeStruct((B, 1, NP), f32),
            jax.ShapeDtypeStruct((B, PN, PN), f32),
        ],
    )(x_enc, rw, rb, wep_tiled, pe2, psel)

    dt, ev = pl.pallas_call(
        _kb_body,
        grid=(B, NTILES),
        in_specs=[
            pl.BlockSpec((1, L, NT), lambda b, j: (b, 0, j)),
            pl.BlockSpec((1, 1, NT), lambda b, j: (b, 0, j)),
            pl.BlockSpec((1, 1, NT), lambda b, j: (b, 0, j)),
            pl.BlockSpec((1, 1, NT), lambda b, j: (0, 0, j)),
            pl.BlockSpec((1, 1, NT), lambda b, j: (0, 0, j)),
            pl.BlockSpec((1, PN, PN), lambda b, j: (b, 0, 0)),
            pl.BlockSpec((PLEN, D2), lambda b, j: (0, 0)),
            pl.BlockSpec((PN, D2), lambda b, j: (0, 0)),
            pl.BlockSpec((D2, D2), lambda b, j: (0, 0)),
            pl.BlockSpec((1, D2), lambda b, j: (0, 0)),
            pl.BlockSpec((D2, D2), lambda b, j: (0, 0)),
            pl.BlockSpec((1, D2), lambda b, j: (0, 0)),
            pl.BlockSpec((PN, D2, DM), lambda b, j: (0, 0, 0)),
            pl.BlockSpec((1, DM), lambda b, j: (0, 0)),
            pl.BlockSpec((DM, PRED), lambda b, j: (0, 0)),
            pl.BlockSpec((1, PRED), lambda b, j: (0, 0)),
            pl.BlockSpec((L, DM), lambda b, j: (0, 0)),
            pl.BlockSpec((1, DM), lambda b, j: (0, 0)),
        ],
        out_specs=[
            pl.BlockSpec((1, NT, PRED), lambda b, j: (b, j, 0)),
            pl.BlockSpec((1, NT, DM), lambda b, j: (b, j, 0)),
        ],
        out_shape=[
            jax.ShapeDtypeStruct((B, NP, PRED), f32),
            jax.ShapeDtypeStruct((B, NP, DM), f32),
        ],
    )(x_enc, mean, std, rw, rb, a_t, W_ep, pe2,
      Wt[0], bt[0].reshape(1, D2), Wt[1], bt[1].reshape(1, D2),
      wsp3, b_sp.reshape(1, DM), W_fc, b_fc.reshape(1, PRED),
      W_ve, b_ve.reshape(1, DM))

    out = pl.pallas_call(
        _kc_body,
        grid=(B,),
        in_specs=[
            pl.BlockSpec((1, NP, DM), lambda b: (b, 0, 0)),
            pl.BlockSpec((1, NP, PRED), lambda b: (b, 0, 0)),
            pl.BlockSpec((DM, DM), lambda b: (0, 0)),
            pl.BlockSpec((1, DM), lambda b: (0, 0)),
            pl.BlockSpec((DM, DM), lambda b: (0, 0)),
            pl.BlockSpec((1, DM), lambda b: (0, 0)),
            pl.BlockSpec((DM, PRED), lambda b: (0, 0)),
            pl.BlockSpec((1, PRED), lambda b: (0, 0)),
            pl.BlockSpec((PRED, PRED), lambda b: (0, 0)),
            pl.BlockSpec((PRED, PRED), lambda b: (0, 0)),
            pl.BlockSpec((1, PRED), lambda b: (0, 0)),
            pl.BlockSpec((1, 1, NP), lambda b: (b, 0, 0)),
            pl.BlockSpec((1, 1, NP), lambda b: (b, 0, 0)),
            pl.BlockSpec((1, 1, NP), lambda b: (0, 0, 0)),
            pl.BlockSpec((1, 1, NP), lambda b: (0, 0, 0)),
        ],
        out_specs=[pl.BlockSpec((1, PRED, NP), lambda b: (b, 0, 0))],
        out_shape=[jax.ShapeDtypeStruct((B, PRED, NP), f32)],
    )(ev, dt, Wv[0], bv[0].reshape(1, DM), Wv[1], bv[1].reshape(1, DM),
      W_fc2, b_fc2.reshape(1, PRED), w3a, w3b, b_fc3.reshape(1, PRED),
      mean, std, rw, rb)[0]

    return out[:, :, :N]


# speedup vs baseline: 6.6704x; 1.1186x over previous
import functools
import math

import jax
import jax.numpy as jnp
import numpy as np
from jax import lax
from jax.experimental import pallas as pl
from jax.experimental.pallas import tpu as pltpu

B = 16
L = 720
N = 862
NP = 896
NT = 128
NTILES = 7
D2 = 64
DM = 128
PN = 45
PLEN = 16
PRED = 96
EPS = 1e-5


def _dot(a, b, ca, cb):
    return lax.dot_general(a, b, (((ca,), (cb,)), ((), ())),
                           preferred_element_type=jnp.float32)


def _top3_softmax(s):
    m1 = jnp.max(s, axis=1, keepdims=True)
    s1 = jnp.where(s >= m1, jnp.float32(-1e30), s)
    m2 = jnp.max(s1, axis=1, keepdims=True)
    s2 = jnp.where(s1 >= m2, jnp.float32(-1e30), s1)
    m3 = jnp.max(s2, axis=1, keepdims=True)
    sm = jnp.where(s >= m3, s, jnp.float32(-1e9))
    e = jnp.exp(sm - m1)
    return e / jnp.sum(e, axis=1, keepdims=True)


def _ka_body(x_ref, rw_ref, rb_ref, wep_ref, pe2_ref, psel_ref,
             wt0_ref, bt0_ref, mean_ref, std_ref, at_ref, pb1_ref):
    xb = x_ref[0]
    col = lax.broadcasted_iota(jnp.int32, (L, NP), 1)
    xb = jnp.where(col < N, xb, 0.0)
    mean = jnp.sum(xb, axis=0, keepdims=True) * (1.0 / L)
    sq = jnp.sum(xb * xb, axis=0, keepdims=True) * (1.0 / L)
    var = sq - mean * mean
    std = jnp.sqrt(var + EPS)
    mean_ref[...] = mean[None]
    std_ref[...] = std[None]
    rw = rw_ref[0]
    rb = rb_ref[0]
    c = rw / (std * N)
    colv = lax.broadcasted_iota(jnp.int32, (1, NP), 1)
    c = jnp.where(colv < N, c, 0.0)
    mcol = jnp.sum(xb * c, axis=1, keepdims=True)
    off = jnp.sum(rb * jnp.where(colv < N, 1.0, 0.0)) * (1.0 / N) \
        - jnp.sum(mean * c)
    z = mcol + off
    zg = z * wep_ref[...]
    q = _dot(psel_ref[...], zg, 1, 0) + pe2_ref[...]
    s = _dot(q, q, 1, 1) * (1.0 / math.sqrt(D2))
    a = _top3_softmax(s)
    at_ref[...] = a[None]
    pb1_ref[...] = (_dot(_dot(a, pe2_ref[...], 1, 0), wt0_ref[...], 1, 0)
                    + bt0_ref[...])[None]


def _kb_body(x_ref, mean_ref, std_ref, rw_ref, rb_ref, at_ref, pb1_ref,
             p16_ref, wc_ref, wt1_ref, bt1_ref,
             wsp_ref, bsp_ref, wfc_ref, bfc_ref, wve_ref, bve_ref,
             dt_ref, ev_ref, a16_ref):
    j = pl.program_id(1)
    @pl.when(j == 0)
    def _build_a16():
        p16 = p16_ref[...]
        t1 = _dot(p16, at_ref[0], 1, 0)
        full = _dot(t1, p16, 1, 1)
        ri = lax.broadcasted_iota(jnp.int32, (L, L), 0)
        ci = lax.broadcasted_iota(jnp.int32, (L, L), 1)
        a16_ref[...] = jnp.where((ri % PLEN) == (ci % PLEN), full, 0.0)

    xb = x_ref[0]
    col = lax.broadcasted_iota(jnp.int32, (L, NT), 1) + j * NT
    mean = mean_ref[0]
    std = std_ref[0]
    xn = (xb - mean) / std * rw_ref[0] + rb_ref[0]
    xn = jnp.where(col < N, xn, 0.0)
    ev = _dot(xn, wve_ref[...], 0, 0) + bve_ref[...]
    ev_ref[...] = ev[None]
    xmix = _dot(a16_ref[...], xn, 1, 0)
    xm3 = xmix.reshape(PN, PLEN, NT)
    pb1 = pb1_ref[0]
    wc = wc_ref[...]
    us = []
    for p in range(PN):
        us.append(jax.nn.gelu(_dot(wc, xm3[p], 0, 0) + pb1[p][:, None]))
    v = jnp.stack(us)
    a = at_ref[0]
    av = _dot(a, v.reshape(PN, D2 * NT), 1, 0).reshape(PN, D2, NT)
    wt1 = wt1_ref[...]
    bt1 = bt1_ref[...]
    u2 = []
    for p in range(PN):
        u2.append(jax.nn.gelu(_dot(wt1, av[p], 0, 0) + bt1[0][:, None]))
    vflat = jnp.concatenate(u2, axis=0)
    dec_time = _dot(vflat, wsp_ref[...], 0, 0) + bsp_ref[...]
    dt = _dot(dec_time, wfc_ref[...], 1, 0) + bfc_ref[...]
    dt_ref[...] = dt[None]


def _kc_body(ev_ref, dt_ref, wv0_ref, bv0_ref, wv1_ref, bv1_ref,
             wfc2_ref, bfc2_ref, w3a_ref, w3b_ref, bfc3_ref,
             mean_ref, std_ref, rw_ref, rb_ref, out_ref):
    ev = ev_ref[0]
    rows = []
    for i in range(NTILES):
        evc = ev[i * NT:(i + 1) * NT]
        sc = _dot(evc, ev, 1, 1) * (1.0 / math.sqrt(DM))
        colm = lax.broadcasted_iota(jnp.int32, (NT, NP), 1)
        sc = jnp.where(colm < N, sc, jnp.float32(-1e9))
        rows.append(_top3_softmax(sc))
    av = jnp.concatenate(rows, axis=0)
    h = ev
    for wv, bv in ((wv0_ref, bv0_ref), (wv1_ref, bv1_ref)):
        h = jax.nn.gelu(_dot(_dot(av, h, 1, 0), wv[...], 1, 0) + bv[...])
    dv = _dot(h, wfc2_ref[...], 1, 0) + bfc2_ref[...]
    dt = dt_ref[0]
    dec = _dot(dt, w3a_ref[...], 1, 0) + _dot(dv, w3b_ref[...], 1, 0) \
        + bfc3_ref[...]
    mean = mean_ref[0, 0][:, None]
    std = std_ref[0, 0][:, None]
    rw = rw_ref[0, 0][:, None]
    rb = rb_ref[0, 0][:, None]
    dec = (dec - rb) / (rw + EPS * EPS) * std + mean
    out_ref[...] = jnp.transpose(dec)[None]


def _pe2_const():
    pos = np.arange(PN)[:, None].astype(np.float32)
    div = np.exp(np.arange(0, D2, 2).astype(np.float32) * -(np.log(10000.0) / D2))
    pe = np.zeros((PN, D2), dtype=np.float32)
    pe[:, 0::2] = np.sin(pos * div)
    pe[:, 1::2] = np.cos(pos * div)
    return pe


@jax.jit
def kernel(x_enc, rev_w, rev_b, W_ep, b_ep, Wt, bt, W_sp, b_sp, W_ve, b_ve,
           Wv, bv, W_fc, b_fc, W_fc2, b_fc2, W_fc3, b_fc3):
    f32 = jnp.float32
    rw = jnp.pad(rev_w, (0, NP - N)).reshape(1, 1, NP)
    rb = jnp.pad(rev_b, (0, NP - N)).reshape(1, 1, NP)
    pe2 = jnp.asarray(_pe2_const()) + b_ep[None, :]
    wep_tiled = jnp.tile(W_ep, (PN, 1))
    psel = jnp.asarray(np.kron(np.eye(PN, dtype=np.float32),
                               np.ones((1, PLEN), np.float32)))
    p16 = jnp.asarray(np.kron(np.eye(PN, dtype=np.float32),
                              np.ones((PLEN, 1), np.float32)))
    wc = W_ep @ Wt[0]
    w3a, w3b = W_fc3[:PRED], W_fc3[PRED:]

    mean, std, a_t, pb1 = pl.pallas_call(
        _ka_body,
        grid=(B,),
        in_specs=[
            pl.BlockSpec((1, L, NP), lambda b: (b, 0, 0)),
            pl.BlockSpec((1, 1, NP), lambda b: (0, 0, 0)),
            pl.BlockSpec((1, 1, NP), lambda b: (0, 0, 0)),
            pl.BlockSpec((L, D2), lambda b: (0, 0)),
            pl.BlockSpec((PN, D2), lambda b: (0, 0)),
            pl.BlockSpec((PN, L), lambda b: (0, 0)),
            pl.BlockSpec((D2, D2), lambda b: (0, 0)),
            pl.BlockSpec((1, D2), lambda b: (0, 0)),
        ],
        out_specs=[
            pl.BlockSpec((1, 1, NP), lambda b: (b, 0, 0)),
            pl.BlockSpec((1, 1, NP), lambda b: (b, 0, 0)),
            pl.BlockSpec((1, PN, PN), lambda b: (b, 0, 0)),
            pl.BlockSpec((1, PN, D2), lambda b: (b, 0, 0)),
        ],
        out_shape=[
            jax.ShapeDtypeStruct((B, 1, NP), f32),
            jax.ShapeDtypeStruct((B, 1, NP), f32),
            jax.ShapeDtypeStruct((B, PN, PN), f32),
            jax.ShapeDtypeStruct((B, PN, D2), f32),
        ],
    )(x_enc, rw, rb, wep_tiled, pe2, psel, Wt[0], bt[0].reshape(1, D2))

    dt, ev = pl.pallas_call(
        _kb_body,
        grid=(B, NTILES),
        in_specs=[
            pl.BlockSpec((1, L, NT), lambda b, j: (b, 0, j)),
            pl.BlockSpec((1, 1, NT), lambda b, j: (b, 0, j)),
            pl.BlockSpec((1, 1, NT), lambda b, j: (b, 0, j)),
            pl.BlockSpec((1, 1, NT), lambda b, j: (0, 0, j)),
            pl.BlockSpec((1, 1, NT), lambda b, j: (0, 0, j)),
            pl.BlockSpec((1, PN, PN), lambda b, j: (b, 0, 0)),
            pl.BlockSpec((1, PN, D2), lambda b, j: (b, 0, 0)),
            pl.BlockSpec((L, PN), lambda b, j: (0, 0)),
            pl.BlockSpec((PLEN, D2), lambda b, j: (0, 0)),
            pl.BlockSpec((D2, D2), lambda b, j: (0, 0)),
            pl.BlockSpec((1, D2), lambda b, j: (0, 0)),
            pl.BlockSpec((PN * D2, DM), lambda b, j: (0, 0)),
            pl.BlockSpec((1, DM), lambda b, j: (0, 0)),
            pl.BlockSpec((DM, PRED), lambda b, j: (0, 0)),
            pl.BlockSpec((1, PRED), lambda b, j: (0, 0)),
            pl.BlockSpec((L, DM), lambda b, j: (0, 0)),
            pl.BlockSpec((1, DM), lambda b, j: (0, 0)),
        ],
        out_specs=[
            pl.BlockSpec((1, NT, PRED), lambda b, j: (b, j, 0)),
            pl.BlockSpec((1, NT, DM), lambda b, j: (b, j, 0)),
        ],
        out_shape=[
            jax.ShapeDtypeStruct((B, NP, PRED), f32),
            jax.ShapeDtypeStruct((B, NP, DM), f32),
        ],
        scratch_shapes=[pltpu.VMEM((L, L), f32)],
    )(x_enc, mean, std, rw, rb, a_t, pb1, p16, wc,
      Wt[1], bt[1].reshape(1, D2),
      W_sp, b_sp.reshape(1, DM), W_fc, b_fc.reshape(1, PRED),
      W_ve, b_ve.reshape(1, DM))

    out = pl.pallas_call(
        _kc_body,
        grid=(B,),
        in_specs=[
            pl.BlockSpec((1, NP, DM), lambda b: (b, 0, 0)),
            pl.BlockSpec((1, NP, PRED), lambda b: (b, 0, 0)),
            pl.BlockSpec((DM, DM), lambda b: (0, 0)),
            pl.BlockSpec((1, DM), lambda b: (0, 0)),
            pl.BlockSpec((DM, DM), lambda b: (0, 0)),
            pl.BlockSpec((1, DM), lambda b: (0, 0)),
            pl.BlockSpec((DM, PRED), lambda b: (0, 0)),
            pl.BlockSpec((1, PRED), lambda b: (0, 0)),
            pl.BlockSpec((PRED, PRED), lambda b: (0, 0)),
            pl.BlockSpec((PRED, PRED), lambda b: (0, 0)),
            pl.BlockSpec((1, PRED), lambda b: (0, 0)),
            pl.BlockSpec((1, 1, NP), lambda b: (b, 0, 0)),
            pl.BlockSpec((1, 1, NP), lambda b: (b, 0, 0)),
            pl.BlockSpec((1, 1, NP), lambda b: (0, 0, 0)),
            pl.BlockSpec((1, 1, NP), lambda b: (0, 0, 0)),
        ],
        out_specs=[pl.BlockSpec((1, PRED, NP), lambda b: (b, 0, 0))],
        out_shape=[jax.ShapeDtypeStruct((B, PRED, NP), f32)],
    )(ev, dt, Wv[0], bv[0].reshape(1, DM), Wv[1], bv[1].reshape(1, DM),
      W_fc2, b_fc2.reshape(1, PRED), w3a, w3b, b_fc3.reshape(1, PRED),
      mean, std, rw, rb)[0]

    return out[:, :, :N]


# speedup vs baseline: 7.5646x; 1.1341x over previous
import functools
import math

import jax
import jax.numpy as jnp
import numpy as np
from jax import lax
from jax.experimental import pallas as pl
from jax.experimental.pallas import tpu as pltpu

B = 16
L = 720
N = 862
NP = 896
NT = 896
NTILES = 1
D2 = 64
DM = 128
PN = 45
PLEN = 16
PRED = 96
EPS = 1e-5


def _dot(a, b, ca, cb):
    return lax.dot_general(a, b, (((ca,), (cb,)), ((), ())),
                           preferred_element_type=jnp.float32)


def _top3_softmax(s):
    m1 = jnp.max(s, axis=1, keepdims=True)
    s1 = jnp.where(s >= m1, jnp.float32(-1e30), s)
    m2 = jnp.max(s1, axis=1, keepdims=True)
    s2 = jnp.where(s1 >= m2, jnp.float32(-1e30), s1)
    m3 = jnp.max(s2, axis=1, keepdims=True)
    sm = jnp.where(s >= m3, s, jnp.float32(-1e9))
    e = jnp.exp(sm - m1)
    return e / jnp.sum(e, axis=1, keepdims=True)


def _ka_body(x_ref, rw_ref, rb_ref, wep_ref, pe2_ref, psel_ref,
             wt0_ref, bt0_ref, mean_ref, std_ref, at_ref, pb1_ref):
    xb = x_ref[0]
    col = lax.broadcasted_iota(jnp.int32, (L, NP), 1)
    xb = jnp.where(col < N, xb, 0.0)
    mean = jnp.sum(xb, axis=0, keepdims=True) * (1.0 / L)
    sq = jnp.sum(xb * xb, axis=0, keepdims=True) * (1.0 / L)
    var = sq - mean * mean
    std = jnp.sqrt(var + EPS)
    mean_ref[...] = mean[None]
    std_ref[...] = std[None]
    rw = rw_ref[0]
    rb = rb_ref[0]
    c = rw / (std * N)
    colv = lax.broadcasted_iota(jnp.int32, (1, NP), 1)
    c = jnp.where(colv < N, c, 0.0)
    mcol = jnp.sum(xb * c, axis=1, keepdims=True)
    off = jnp.sum(rb * jnp.where(colv < N, 1.0, 0.0)) * (1.0 / N) \
        - jnp.sum(mean * c)
    z = mcol + off
    zg = z * wep_ref[...]
    q = _dot(psel_ref[...], zg, 1, 0) + pe2_ref[...]
    s = _dot(q, q, 1, 1) * (1.0 / math.sqrt(D2))
    a = _top3_softmax(s)
    at_ref[...] = a[None]
    pb1_ref[...] = (_dot(_dot(a, pe2_ref[...], 1, 0), wt0_ref[...], 1, 0)
                    + bt0_ref[...])[None]


def _kb_body(x_ref, mean_ref, std_ref, rw_ref, rb_ref, at_ref, pb1_ref,
             p16_ref, wc_ref, wt1_ref, bt1_ref,
             wsp_ref, bsp_ref, wfc_ref, bfc_ref, wve_ref, bve_ref,
             dt_ref, ev_ref, a16_ref):
    j = pl.program_id(1)
    @pl.when(j == 0)
    def _build_a16():
        p16 = p16_ref[...]
        t1 = _dot(p16, at_ref[0], 1, 0)
        full = _dot(t1, p16, 1, 1)
        ri = lax.broadcasted_iota(jnp.int32, (L, L), 0)
        ci = lax.broadcasted_iota(jnp.int32, (L, L), 1)
        a16_ref[...] = jnp.where((ri % PLEN) == (ci % PLEN), full, 0.0)

    xb = x_ref[0]
    col = lax.broadcasted_iota(jnp.int32, (L, NT), 1) + j * NT
    mean = mean_ref[0]
    std = std_ref[0]
    xn = (xb - mean) / std * rw_ref[0] + rb_ref[0]
    xn = jnp.where(col < N, xn, 0.0)
    ev = _dot(xn, wve_ref[...], 0, 0) + bve_ref[...]
    ev_ref[...] = ev[None]
    xmix = _dot(a16_ref[...], xn, 1, 0)
    xm3 = xmix.reshape(PN, PLEN, NT)
    pb1 = pb1_ref[0]
    wc = wc_ref[...]
    us = []
    for p in range(PN):
        us.append(jax.nn.gelu(_dot(wc, xm3[p], 0, 0) + pb1[p][:, None]))
    v = jnp.stack(us)
    a = at_ref[0]
    av = _dot(a, v.reshape(PN, D2 * NT), 1, 0).reshape(PN, D2, NT)
    wt1 = wt1_ref[...]
    bt1 = bt1_ref[...]
    u2 = []
    for p in range(PN):
        u2.append(jax.nn.gelu(_dot(wt1, av[p], 0, 0) + bt1[0][:, None]))
    vflat = jnp.concatenate(u2, axis=0)
    dec_time = _dot(vflat, wsp_ref[...], 0, 0) + bsp_ref[...]
    dt = _dot(dec_time, wfc_ref[...], 1, 0) + bfc_ref[...]
    dt_ref[...] = dt[None]


def _kc_body(ev_ref, dt_ref, wv0_ref, bv0_ref, wv1_ref, bv1_ref,
             wfc2_ref, bfc2_ref, w3a_ref, w3b_ref, bfc3_ref,
             mean_ref, std_ref, rw_ref, rb_ref, out_ref):
    ev = ev_ref[0]
    rows = []
    for i in range(7):
        evc = ev[i * 128:(i + 1) * 128]
        sc = _dot(evc, ev, 1, 1) * (1.0 / math.sqrt(DM))
        colm = lax.broadcasted_iota(jnp.int32, (128, NP), 1)
        sc = jnp.where(colm < N, sc, jnp.float32(-1e9))
        rows.append(_top3_softmax(sc))
    av = jnp.concatenate(rows, axis=0)
    h = ev
    for wv, bv in ((wv0_ref, bv0_ref), (wv1_ref, bv1_ref)):
        h = jax.nn.gelu(_dot(_dot(av, h, 1, 0), wv[...], 1, 0) + bv[...])
    dv = _dot(h, wfc2_ref[...], 1, 0) + bfc2_ref[...]
    dt = dt_ref[0]
    dec = _dot(dt, w3a_ref[...], 1, 0) + _dot(dv, w3b_ref[...], 1, 0) \
        + bfc3_ref[...]
    mean = mean_ref[0, 0][:, None]
    std = std_ref[0, 0][:, None]
    rw = rw_ref[0, 0][:, None]
    rb = rb_ref[0, 0][:, None]
    dec = (dec - rb) / (rw + EPS * EPS) * std + mean
    out_ref[...] = jnp.transpose(dec)[None]


def _pe2_const():
    pos = np.arange(PN)[:, None].astype(np.float32)
    div = np.exp(np.arange(0, D2, 2).astype(np.float32) * -(np.log(10000.0) / D2))
    pe = np.zeros((PN, D2), dtype=np.float32)
    pe[:, 0::2] = np.sin(pos * div)
    pe[:, 1::2] = np.cos(pos * div)
    return pe


@jax.jit
def kernel(x_enc, rev_w, rev_b, W_ep, b_ep, Wt, bt, W_sp, b_sp, W_ve, b_ve,
           Wv, bv, W_fc, b_fc, W_fc2, b_fc2, W_fc3, b_fc3):
    f32 = jnp.float32
    rw = jnp.pad(rev_w, (0, NP - N)).reshape(1, 1, NP)
    rb = jnp.pad(rev_b, (0, NP - N)).reshape(1, 1, NP)
    pe2 = jnp.asarray(_pe2_const()) + b_ep[None, :]
    wep_tiled = jnp.tile(W_ep, (PN, 1))
    psel = jnp.asarray(np.kron(np.eye(PN, dtype=np.float32),
                               np.ones((1, PLEN), np.float32)))
    p16 = jnp.asarray(np.kron(np.eye(PN, dtype=np.float32),
                              np.ones((PLEN, 1), np.float32)))
    wc = W_ep @ Wt[0]
    w3a, w3b = W_fc3[:PRED], W_fc3[PRED:]

    mean, std, a_t, pb1 = pl.pallas_call(
        _ka_body,
        grid=(B,),
        in_specs=[
            pl.BlockSpec((1, L, NP), lambda b: (b, 0, 0)),
            pl.BlockSpec((1, 1, NP), lambda b: (0, 0, 0)),
            pl.BlockSpec((1, 1, NP), lambda b: (0, 0, 0)),
            pl.BlockSpec((L, D2), lambda b: (0, 0)),
            pl.BlockSpec((PN, D2), lambda b: (0, 0)),
            pl.BlockSpec((PN, L), lambda b: (0, 0)),
            pl.BlockSpec((D2, D2), lambda b: (0, 0)),
            pl.BlockSpec((1, D2), lambda b: (0, 0)),
        ],
        out_specs=[
            pl.BlockSpec((1, 1, NP), lambda b: (b, 0, 0)),
            pl.BlockSpec((1, 1, NP), lambda b: (b, 0, 0)),
            pl.BlockSpec((1, PN, PN), lambda b: (b, 0, 0)),
            pl.BlockSpec((1, PN, D2), lambda b: (b, 0, 0)),
        ],
        out_shape=[
            jax.ShapeDtypeStruct((B, 1, NP), f32),
            jax.ShapeDtypeStruct((B, 1, NP), f32),
            jax.ShapeDtypeStruct((B, PN, PN), f32),
            jax.ShapeDtypeStruct((B, PN, D2), f32),
        ],
    )(x_enc, rw, rb, wep_tiled, pe2, psel, Wt[0], bt[0].reshape(1, D2))

    dt, ev = pl.pallas_call(
        _kb_body,
        grid=(B, NTILES),
        in_specs=[
            pl.BlockSpec((1, L, NT), lambda b, j: (b, 0, j)),
            pl.BlockSpec((1, 1, NT), lambda b, j: (b, 0, j)),
            pl.BlockSpec((1, 1, NT), lambda b, j: (b, 0, j)),
            pl.BlockSpec((1, 1, NT), lambda b, j: (0, 0, j)),
            pl.BlockSpec((1, 1, NT), lambda b, j: (0, 0, j)),
            pl.BlockSpec((1, PN, PN), lambda b, j: (b, 0, 0)),
            pl.BlockSpec((1, PN, D2), lambda b, j: (b, 0, 0)),
            pl.BlockSpec((L, PN), lambda b, j: (0, 0)),
            pl.BlockSpec((PLEN, D2), lambda b, j: (0, 0)),
            pl.BlockSpec((D2, D2), lambda b, j: (0, 0)),
            pl.BlockSpec((1, D2), lambda b, j: (0, 0)),
            pl.BlockSpec((PN * D2, DM), lambda b, j: (0, 0)),
            pl.BlockSpec((1, DM), lambda b, j: (0, 0)),
            pl.BlockSpec((DM, PRED), lambda b, j: (0, 0)),
            pl.BlockSpec((1, PRED), lambda b, j: (0, 0)),
            pl.BlockSpec((L, DM), lambda b, j: (0, 0)),
            pl.BlockSpec((1, DM), lambda b, j: (0, 0)),
        ],
        out_specs=[
            pl.BlockSpec((1, NT, PRED), lambda b, j: (b, j, 0)),
            pl.BlockSpec((1, NT, DM), lambda b, j: (b, j, 0)),
        ],
        out_shape=[
            jax.ShapeDtypeStruct((B, NP, PRED), f32),
            jax.ShapeDtypeStruct((B, NP, DM), f32),
        ],
        scratch_shapes=[pltpu.VMEM((L, L), f32)],
    )(x_enc, mean, std, rw, rb, a_t, pb1, p16, wc,
      Wt[1], bt[1].reshape(1, D2),
      W_sp, b_sp.reshape(1, DM), W_fc, b_fc.reshape(1, PRED),
      W_ve, b_ve.reshape(1, DM))

    out = pl.pallas_call(
        _kc_body,
        grid=(B,),
        in_specs=[
            pl.BlockSpec((1, NP, DM), lambda b: (b, 0, 0)),
            pl.BlockSpec((1, NP, PRED), lambda b: (b, 0, 0)),
            pl.BlockSpec((DM, DM), lambda b: (0, 0)),
            pl.BlockSpec((1, DM), lambda b: (0, 0)),
            pl.BlockSpec((DM, DM), lambda b: (0, 0)),
            pl.BlockSpec((1, DM), lambda b: (0, 0)),
            pl.BlockSpec((DM, PRED), lambda b: (0, 0)),
            pl.BlockSpec((1, PRED), lambda b: (0, 0)),
            pl.BlockSpec((PRED, PRED), lambda b: (0, 0)),
            pl.BlockSpec((PRED, PRED), lambda b: (0, 0)),
            pl.BlockSpec((1, PRED), lambda b: (0, 0)),
            pl.BlockSpec((1, 1, NP), lambda b: (b, 0, 0)),
            pl.BlockSpec((1, 1, NP), lambda b: (b, 0, 0)),
            pl.BlockSpec((1, 1, NP), lambda b: (0, 0, 0)),
            pl.BlockSpec((1, 1, NP), lambda b: (0, 0, 0)),
        ],
        out_specs=[pl.BlockSpec((1, PRED, NP), lambda b: (b, 0, 0))],
        out_shape=[jax.ShapeDtypeStruct((B, PRED, NP), f32)],
    )(ev, dt, Wv[0], bv[0].reshape(1, DM), Wv[1], bv[1].reshape(1, DM),
      W_fc2, b_fc2.reshape(1, PRED), w3a, w3b, b_fc3.reshape(1, PRED),
      mean, std, rw, rb)[0]

    return out[:, :, :N]


# speedup vs baseline: 7.9334x; 1.0487x over previous
import math

import jax
import jax.numpy as jnp
import numpy as np
from jax import lax
from jax.experimental import pallas as pl

B = 16
L = 720
N = 862
NP = 896
D2 = 64
DM = 128
PN = 45
PLEN = 16
PRED = 96
EPS = 1e-5


def _dot(a, b, ca, cb):
    return lax.dot_general(a, b, (((ca,), (cb,)), ((), ())),
                           preferred_element_type=jnp.float32)


def _top3_softmax(s):
    m1 = jnp.max(s, axis=1, keepdims=True)
    s1 = jnp.where(s >= m1, jnp.float32(-1e30), s)
    m2 = jnp.max(s1, axis=1, keepdims=True)
    s2 = jnp.where(s1 >= m2, jnp.float32(-1e30), s1)
    m3 = jnp.max(s2, axis=1, keepdims=True)
    sm = jnp.where(s >= m3, s, jnp.float32(-1e9))
    e = jnp.exp(sm - m1)
    return e / jnp.sum(e, axis=1, keepdims=True)


def _body(x_ref, rw_ref, rb_ref, wep_ref, pe2_ref, psel_ref, wt0_ref, bt0_ref,
          p16_ref, wc_ref, wt1_ref, bt1_ref, wsp_ref, bsp_ref, wfc_ref,
          bfc_ref, wve_ref, bve_ref, wv0_ref, bv0_ref, wv1_ref, bv1_ref,
          wfc2_ref, bfc2_ref, w3a_ref, w3b_ref, bfc3_ref, out_ref):
    xb = x_ref[0]
    col = lax.broadcasted_iota(jnp.int32, (L, NP), 1)
    xb = jnp.where(col < N, xb, 0.0)
    mean = jnp.sum(xb, axis=0, keepdims=True) * (1.0 / L)
    sq = jnp.sum(xb * xb, axis=0, keepdims=True) * (1.0 / L)
    std = jnp.sqrt(sq - mean * mean + EPS)
    rw = rw_ref[0]
    rb = rb_ref[0]
    c = rw / (std * N)
    colv = lax.broadcasted_iota(jnp.int32, (1, NP), 1)
    c = jnp.where(colv < N, c, 0.0)
    mcol = jnp.sum(xb * c, axis=1, keepdims=True)
    off = jnp.sum(rb * jnp.where(colv < N, 1.0, 0.0)) * (1.0 / N) \
        - jnp.sum(mean * c)
    z = mcol + off
    zg = z * wep_ref[...]
    q = _dot(psel_ref[...], zg, 1, 0) + pe2_ref[...]
    s = _dot(q, q, 1, 1) * (1.0 / math.sqrt(D2))
    a = _top3_softmax(s)
    pb1 = _dot(_dot(a, pe2_ref[...], 1, 0), wt0_ref[...], 1, 0) + bt0_ref[...]
    p16 = p16_ref[...]
    full = _dot(_dot(p16, a, 1, 0), p16, 1, 1)
    ri = lax.broadcasted_iota(jnp.int32, (L, L), 0)
    ci = lax.broadcasted_iota(jnp.int32, (L, L), 1)
    a16 = jnp.where((ri % PLEN) == (ci % PLEN), full, 0.0)
    xn = (xb - mean) / std * rw + rb
    xn = jnp.where(col < N, xn, 0.0)
    ev = _dot(xn, wve_ref[...], 0, 0) + bve_ref[...]
    xmix = _dot(a16, xn, 1, 0)
    xm3 = xmix.reshape(PN, PLEN, NP)
    wc = wc_ref[...]
    us = []
    for p in range(PN):
        us.append(jax.nn.gelu(_dot(wc, xm3[p], 0, 0) + pb1[p][:, None]))
    v = jnp.stack(us)
    av = _dot(a, v.reshape(PN, D2 * NP), 1, 0).reshape(PN, D2, NP)
    wt1 = wt1_ref[...]
    bt1 = bt1_ref[...]
    u2 = []
    for p in range(PN):
        u2.append(jax.nn.gelu(_dot(wt1, av[p], 0, 0) + bt1[0][:, None]))
    vflat = jnp.concatenate(u2, axis=0)
    dec_time = _dot(vflat, wsp_ref[...], 0, 0) + bsp_ref[...]
    dt = _dot(dec_time, wfc_ref[...], 1, 0) + bfc_ref[...]
    rows = []
    for i in range(7):
        evc = ev[i * 128:(i + 1) * 128]
        sc = _dot(evc, ev, 1, 1) * (1.0 / math.sqrt(DM))
        colm = lax.broadcasted_iota(jnp.int32, (128, NP), 1)
        sc = jnp.where(colm < N, sc, jnp.float32(-1e9))
        rows.append(_top3_softmax(sc))
    avar = jnp.concatenate(rows, axis=0)
    h = ev
    for wv, bv in ((wv0_ref, bv0_ref), (wv1_ref, bv1_ref)):
        h = jax.nn.gelu(_dot(_dot(avar, h, 1, 0), wv[...], 1, 0) + bv[...])
    dv = _dot(h, wfc2_ref[...], 1, 0) + bfc2_ref[...]
    dec = _dot(dt, w3a_ref[...], 1, 0) + _dot(dv, w3b_ref[...], 1, 0) \
        + bfc3_ref[...]
    dec = (dec - rb[0][:, None]) / (rw[0][:, None] + EPS * EPS) \
        * std[0][:, None] + mean[0][:, None]
    out_ref[...] = jnp.transpose(dec)[None]


def _pe2_const():
    pos = np.arange(PN)[:, None].astype(np.float32)
    div = np.exp(np.arange(0, D2, 2).astype(np.float32) * -(np.log(10000.0) / D2))
    pe = np.zeros((PN, D2), dtype=np.float32)
    pe[:, 0::2] = np.sin(pos * div)
    pe[:, 1::2] = np.cos(pos * div)
    return pe


@jax.jit
def kernel(x_enc, rev_w, rev_b, W_ep, b_ep, Wt, bt, W_sp, b_sp, W_ve, b_ve,
           Wv, bv, W_fc, b_fc, W_fc2, b_fc2, W_fc3, b_fc3):
    f32 = jnp.float32
    rw = jnp.pad(rev_w, (0, NP - N)).reshape(1, 1, NP)
    rb = jnp.pad(rev_b, (0, NP - N)).reshape(1, 1, NP)
    pe2 = jnp.asarray(_pe2_const()) + b_ep[None, :]
    wep_tiled = jnp.tile(W_ep, (PN, 1))
    psel = jnp.asarray(np.kron(np.eye(PN, dtype=np.float32),
                               np.ones((1, PLEN), np.float32)))
    p16 = jnp.asarray(np.kron(np.eye(PN, dtype=np.float32),
                              np.ones((PLEN, 1), np.float32)))
    wc = W_ep @ Wt[0]
    w3a, w3b = W_fc3[:PRED], W_fc3[PRED:]

    def w(shape):
        return pl.BlockSpec(shape, lambda b: tuple(0 for _ in shape))

    out = pl.pallas_call(
        _body,
        grid=(B,),
        in_specs=[
            pl.BlockSpec((1, L, NP), lambda b: (b, 0, 0)),
            w((1, 1, NP)), w((1, 1, NP)),
            w((L, D2)), w((PN, D2)), w((PN, L)), w((D2, D2)), w((1, D2)),
            w((L, PN)), w((PLEN, D2)), w((D2, D2)), w((1, D2)),
            w((PN * D2, DM)), w((1, DM)), w((DM, PRED)), w((1, PRED)),
            w((L, DM)), w((1, DM)),
            w((DM, DM)), w((1, DM)), w((DM, DM)), w((1, DM)),
            w((DM, PRED)), w((1, PRED)),
            w((PRED, PRED)), w((PRED, PRED)), w((1, PRED)),
        ],
        out_specs=[pl.BlockSpec((1, PRED, NP), lambda b: (b, 0, 0))],
        out_shape=[jax.ShapeDtypeStruct((B, PRED, NP), f32)],
    )(x_enc, rw, rb, wep_tiled, pe2, psel, Wt[0], bt[0].reshape(1, D2),
      p16, wc, Wt[1], bt[1].reshape(1, D2),
      W_sp, b_sp.reshape(1, DM), W_fc, b_fc.reshape(1, PRED),
      W_ve, b_ve.reshape(1, DM),
      Wv[0], bv[0].reshape(1, DM), Wv[1], bv[1].reshape(1, DM),
      W_fc2, b_fc2.reshape(1, PRED), w3a, w3b, b_fc3.reshape(1, PRED))[0]

    return out[:, :, :N]
